# baseline (speedup 1.0000x reference)
import functools

import jax
import jax.numpy as jnp
from jax import lax
from jax.experimental import pallas as pl
from jax.experimental.pallas import tpu as pltpu
from jax.experimental.pallas import tpu_sc as plsc

NUM_EMB = 1_000_000
OUT_DIM = 32
BATCH = 16384
QD = 4
SD = 8
WIN = 16

_info = plsc.get_sparse_core_info()
_NC, _NS, _NL = _info.num_cores, _info.num_subcores, _info.num_lanes
_NW = _NC * _NS
_B_PER_W = BATCH // _NW
_SLAB = 128
_N_SLAB = _B_PER_W // _SLAB
_NG = _SLAB // _NL


_mesh = plsc.VectorSubcoreMesh(core_axis_name="c", subcore_axis_name="s")


@functools.partial(
    pl.kernel,
    mesh=_mesh,
    out_type=jax.ShapeDtypeStruct((QD, SD, BATCH), jnp.float32),
    scratch_types=[
        pltpu.VMEM((_B_PER_W + _NL,), jnp.int32),
        pltpu.VMEM((QD, SD, _SLAB * WIN), jnp.float32),
        pltpu.VMEM((QD, SD, _SLAB), jnp.float32),
        pltpu.SemaphoreType.DMA,
    ],
    compiler_params=pltpu.CompilerParams(needs_layout_passes=False),
)
def _gather_kernel(idx_hbm, table_hbm, out_hbm, idx_v, win_v, sel_v, sem):
    wid = lax.axis_index("s") * _NC + lax.axis_index("c")
    base = wid * _B_PER_W
    pltpu.sync_copy(idx_hbm.at[pl.ds(base, _B_PER_W)], idx_v.at[pl.ds(0, _B_PER_W)])

    def fire_group(goff):
        @pl.loop(0, _NL)
        def _fire(j):
            p = goff + j
            r = idx_v[pl.ds(p, _NL)][0]
            a128 = pl.multiple_of(r & ~127, 128)
            sub = lax.shift_right_logical(r, 4) & 7
            pw = (p % _SLAB) * WIN

            def leaf(k):
                pltpu.async_copy(
                    table_hbm.at[:, :, pl.ds(a128, 128)].at[
                        :, :, pl.ds(k * WIN, WIN)
                    ],
                    win_v.at[:, :, pl.ds(pw, WIN)],
                    sem,
                )

            @pl.when(sub < 4)
            def _lo():
                @pl.when(sub < 2)
                def _a():
                    @pl.when(sub == 0)
                    def _k0():
                        leaf(0)

                    @pl.when(sub == 1)
                    def _k1():
                        leaf(1)

                @pl.when(sub >= 2)
                def _b():
                    @pl.when(sub == 2)
                    def _k2():
                        leaf(2)

                    @pl.when(sub == 3)
                    def _k3():
                        leaf(3)

            @pl.when(sub >= 4)
            def _hi():
                @pl.when(sub < 6)
                def _c():
                    @pl.when(sub == 4)
                    def _k4():
                        leaf(4)

                    @pl.when(sub == 5)
                    def _k5():
                        leaf(5)

                @pl.when(sub >= 6)
                def _d():
                    @pl.when(sub == 6)
                    def _k6():
                        leaf(6)

                    @pl.when(sub == 7)
                    def _k7():
                        leaf(7)

    def drain_group():
        for _ in range(_NL):
            pltpu.make_async_copy(
                table_hbm.at[:, :, pl.ds(0, WIN)],
                win_v.at[:, :, pl.ds(0, WIN)],
                sem,
            ).wait()

    def select_group(off, g):
        gpos = off + g * _NL
        idx_vec = idx_v[pl.ds(gpos, _NL)]
        j_vec = jnp.arange(_NL, dtype=jnp.int32) + g * _NL
        col_vec = j_vec * WIN + (idx_vec & (WIN - 1))
        for q in range(QD):
            q_vec = jnp.full((_NL,), q, dtype=jnp.int32)
            for s in range(SD):
                s_vec = jnp.full((_NL,), s, dtype=jnp.int32)
                vals = plsc.load_gather(win_v, [q_vec, s_vec, col_vec])
                plsc.store_scatter(sel_v, [q_vec, s_vec, j_vec], vals)

    @pl.loop(0, _N_SLAB)
    def _slab(w):
        off = w * _SLAB
        fire_group(off)
        fire_group(off + _NL)

        @pl.loop(0, _NG)
        def _grp(g):
            @pl.when(g < _NG - 2)
            def _next():
                fire_group(off + (g + 2) * _NL)

            drain_group()
            select_group(off, g)

        pltpu.sync_copy(sel_v, out_hbm.at[:, :, pl.ds(base + off, _SLAB)])


def kernel(x, table):
    idx = x.reshape(BATCH).astype(jnp.int32)
    table3 = table.T.reshape(QD, SD, NUM_EMB)
    out3 = _gather_kernel(idx, table3)
    return out3.reshape(OUT_DIM, BATCH).T


# speedup vs baseline: 1.0193x; 1.0193x over previous
import functools

import jax
import jax.numpy as jnp
from jax import lax
from jax.experimental import pallas as pl
from jax.experimental.pallas import tpu as pltpu
from jax.experimental.pallas import tpu_sc as plsc

NUM_EMB = 1_000_000
OUT_DIM = 32
BATCH = 16384
QD = 4
SD = 8
WIN = 16

_info = plsc.get_sparse_core_info()
_NC, _NS, _NL = _info.num_cores, _info.num_subcores, _info.num_lanes
_NW = _NC * _NS
_B_PER_W = BATCH // _NW
_SLAB = 128
_N_SLAB = _B_PER_W // _SLAB
_NG = _SLAB // _NL


_mesh = plsc.VectorSubcoreMesh(core_axis_name="c", subcore_axis_name="s")


@functools.partial(
    pl.kernel,
    mesh=_mesh,
    out_type=jax.ShapeDtypeStruct((QD, SD, BATCH), jnp.float32),
    scratch_types=[
        pltpu.VMEM((_B_PER_W + _NL,), jnp.int32),
        pltpu.VMEM((QD, SD, _SLAB * WIN), jnp.float32),
        pltpu.VMEM((QD, SD, _SLAB), jnp.float32),
        pltpu.SemaphoreType.DMA,
    ],
    compiler_params=pltpu.CompilerParams(needs_layout_passes=False),
)
def _gather_kernel(idx_hbm, table_hbm, out_hbm, idx_v, win_v, sel_v, sem):
    wid = lax.axis_index("s") * _NC + lax.axis_index("c")
    base = wid * _B_PER_W
    pltpu.sync_copy(idx_hbm.at[pl.ds(base, _B_PER_W)], idx_v.at[pl.ds(0, _B_PER_W)])

    def fire_group(goff):
        @pl.loop(0, _NL)
        def _fire(j):
            p = goff + j
            r = idx_v[pl.ds(p, _NL)][0]
            a128 = pl.multiple_of(r & ~127, 128)
            sub = lax.shift_right_logical(r, 4) & 7
            pw = (p % _SLAB) * WIN

            def leaf(k):
                pltpu.async_copy(
                    table_hbm.at[:, :, pl.ds(a128, 128)].at[
                        :, :, pl.ds(k * WIN, WIN)
                    ],
                    win_v.at[:, :, pl.ds(pw, WIN)],
                    sem,
                )

            @pl.when(sub < 4)
            def _lo():
                @pl.when(sub < 2)
                def _a():
                    @pl.when(sub == 0)
                    def _k0():
                        leaf(0)

                    @pl.when(sub == 1)
                    def _k1():
                        leaf(1)

                @pl.when(sub >= 2)
                def _b():
                    @pl.when(sub == 2)
                    def _k2():
                        leaf(2)

                    @pl.when(sub == 3)
                    def _k3():
                        leaf(3)

            @pl.when(sub >= 4)
            def _hi():
                @pl.when(sub < 6)
                def _c():
                    @pl.when(sub == 4)
                    def _k4():
                        leaf(4)

                    @pl.when(sub == 5)
                    def _k5():
                        leaf(5)

                @pl.when(sub >= 6)
                def _d():
                    @pl.when(sub == 6)
                    def _k6():
                        leaf(6)

                    @pl.when(sub == 7)
                    def _k7():
                        leaf(7)

    def drain_group():
        for _ in range(_NL):
            pltpu.make_async_copy(
                table_hbm.at[:, :, pl.ds(0, WIN)],
                win_v.at[:, :, pl.ds(0, WIN)],
                sem,
            ).wait()

    def select_group(off, g):
        gpos = off + g * _NL
        idx_vec = idx_v[pl.ds(gpos, _NL)]
        j_vec = jnp.arange(_NL, dtype=jnp.int32) + g * _NL
        col_vec = j_vec * WIN + (idx_vec & (WIN - 1))
        for q in range(QD):
            q_vec = jnp.full((_NL,), q, dtype=jnp.int32)
            for s in range(SD):
                s_vec = jnp.full((_NL,), s, dtype=jnp.int32)
                vals = plsc.load_gather(win_v, [q_vec, s_vec, col_vec])
                plsc.store_scatter(sel_v, [q_vec, s_vec, j_vec], vals)

    @pl.loop(0, _N_SLAB)
    def _slab(w):
        off = w * _SLAB
        fire_group(off)

        @pl.loop(0, _NG)
        def _grp(g):
            @pl.when(g < _NG - 1)
            def _next():
                fire_group(off + (g + 1) * _NL)

            drain_group()
            select_group(off, g)

        pltpu.sync_copy(sel_v, out_hbm.at[:, :, pl.ds(base + off, _SLAB)])


def kernel(x, table):
    idx = x.reshape(BATCH).astype(jnp.int32)
    table3 = table.T.reshape(QD, SD, NUM_EMB)
    out3 = _gather_kernel(idx, table3)
    return out3.reshape(OUT_DIM, BATCH).T
